# baseline (speedup 1.0000x reference)
import functools
import math

import jax
import jax.numpy as jnp
from jax import lax
from jax.experimental import pallas as pl
from jax.experimental.pallas import tpu as pltpu
from jax.experimental.pallas import tpu_sc as plsc

_NUM_CLASSES = 100000
_BETA = 0.8
_ROWS = 1024
_BR = 2000
_Q = _BR // 8
_NBLK = _NUM_CLASSES // _BR
_LN_BETA = math.log(_BETA)
_LOG2E = math.log2(math.e)
_NEG_INF = float("-inf")


def _loss_kernel(x_ref, trow_ref, out_ref, m_ref, s_ref, p_ref, trel_ref):
    j = pl.program_id(0)

    @pl.when(j == 0)
    def _init():
        m_ref[...] = jnp.full((8, _ROWS), _NEG_INF, jnp.float32)
        s_ref[...] = jnp.zeros((8, _ROWS), jnp.float32)
        p_ref[...] = jnp.zeros((8, _ROWS), jnp.float32)
        sub = jax.lax.broadcasted_iota(jnp.int32, (8, _ROWS), 0)
        trel_ref[...] = trow_ref[...] - sub

    x3 = x_ref[...].reshape(_Q, 8, _ROWS)
    tm = jnp.max(x3, axis=0)
    new_m = jnp.maximum(m_ref[...], tm)
    alpha = jnp.exp(m_ref[...] - new_m)
    nm2 = new_m * (-_LOG2E)
    acc = jnp.sum(jnp.exp2(x3 * _LOG2E + nm2[None]), axis=0)
    s_ref[...] = s_ref[...] * alpha + acc
    m_ref[...] = new_m

    u = trel_ref[...] - j * _BR
    p = p_ref[...]
    for q in range(_Q):
        p = jnp.where(u == 8 * q, x3[q], p)
    p_ref[...] = p

    @pl.when(j == _NBLK - 1)
    def _fin():
        m_row = jnp.max(m_ref[...], axis=0, keepdims=True)
        s_row = jnp.sum(s_ref[...] * jnp.exp(m_ref[...] - m_row),
                        axis=0, keepdims=True)
        picked = jnp.sum(p_ref[...], axis=0, keepdims=True)
        nll = m_row + jnp.log(s_row) - picked
        out_ref[...] = jnp.sum(nll, keepdims=True) / _ROWS


def _loss(x_t, trow):
    return pl.pallas_call(
        _loss_kernel,
        grid=(_NBLK,),
        in_specs=[
            pl.BlockSpec((_BR, _ROWS), lambda j: (j, 0)),
            pl.BlockSpec((1, _ROWS), lambda j: (0, 0)),
        ],
        out_specs=pl.BlockSpec((1, 1), lambda j: (0, 0)),
        out_shape=jax.ShapeDtypeStruct((1, 1), jnp.float32),
        scratch_shapes=[
            pltpu.VMEM((8, _ROWS), jnp.float32),
            pltpu.VMEM((8, _ROWS), jnp.float32),
            pltpu.VMEM((8, _ROWS), jnp.float32),
            pltpu.VMEM((8, _ROWS), jnp.int32),
        ],
    )(x_t, trow)


_NW = 32
_TPW = _ROWS // _NW


def _sc_weights_body(t2_hbm, out_hbm, t2_v, psum_v):
    cid = lax.axis_index("c")
    sid = lax.axis_index("s")
    wid = sid * 2 + cid
    pltpu.sync_copy(t2_hbm, t2_v)
    base = wid * _TPW
    tg1 = t2_v[pl.ds(base, 16)]
    tg2 = t2_v[pl.ds(base + 16, 16)]

    def body(off, cs):
        c1, c2 = cs
        ts = t2_v[pl.ds(off, 16)]
        one = jnp.int32(1)
        zero = jnp.int32(0)
        c1 = c1 + jnp.where(tg1 == ts, one, zero)
        c2 = c2 + jnp.where(tg2 == ts, one, zero)
        return (c1, c2)

    z = jnp.zeros((16,), jnp.int32)
    c1, c2 = lax.fori_loop(0, _ROWS, body, (z, z))

    def wsum(c):
        cf = c.astype(jnp.float32)
        return (1.0 - jnp.exp(cf * (_LN_BETA / _ROWS))) / cf

    psum_v[...] = wsum(c1) + wsum(c2)
    pltpu.sync_copy(psum_v, out_hbm.at[wid])


@functools.lru_cache(maxsize=None)
def _make_sc_weights():
    return pl.kernel(
        _sc_weights_body,
        mesh=plsc.VectorSubcoreMesh(core_axis_name="c", subcore_axis_name="s"),
        out_type=jax.ShapeDtypeStruct((_NW, 16), jnp.float32),
        scratch_types=[
            pltpu.VMEM((2 * _ROWS,), jnp.int32),
            pltpu.VMEM((16,), jnp.float32),
        ],
    )


def _sc_weights(t32):
    return _make_sc_weights()(jnp.concatenate([t32, t32]))


def kernel(inputs, targets):
    t32 = targets.astype(jnp.int32)
    loss = _loss(inputs.T, t32.reshape(1, _ROWS))[0, 0]
    wparts = _sc_weights(t32)
    return loss * (jnp.sum(wparts) / _NUM_CLASSES)


# speedup vs baseline: 1.0299x; 1.0299x over previous
import functools
import math

import jax
import jax.numpy as jnp
from jax import lax
from jax.experimental import pallas as pl
from jax.experimental.pallas import tpu as pltpu
from jax.experimental.pallas import tpu_sc as plsc

_NUM_CLASSES = 100000
_BETA = 0.8
_ROWS = 1024
_BR = 4000
_Q = _BR // 8
_NBLK = _NUM_CLASSES // _BR
_LN_BETA = math.log(_BETA)
_LOG2E = math.log2(math.e)
_NEG_INF = float("-inf")


def _loss_kernel(x_ref, trow_ref, out_ref, m_ref, s_ref, p_ref, trel_ref):
    j = pl.program_id(0)

    @pl.when(j == 0)
    def _init():
        m_ref[...] = jnp.full((8, _ROWS), _NEG_INF, jnp.float32)
        s_ref[...] = jnp.zeros((8, _ROWS), jnp.float32)
        p_ref[...] = jnp.zeros((8, _ROWS), jnp.float32)
        sub = jax.lax.broadcasted_iota(jnp.int32, (8, _ROWS), 0)
        trel_ref[...] = trow_ref[...] - sub

    x3 = x_ref[...].reshape(_Q, 8, _ROWS)
    tm = jnp.max(x3, axis=0)
    new_m = jnp.maximum(m_ref[...], tm)
    alpha = jnp.exp(m_ref[...] - new_m)
    nm2 = new_m * (-_LOG2E)
    acc = jnp.sum(jnp.exp2(x3 * _LOG2E + nm2[None]), axis=0)
    s_ref[...] = s_ref[...] * alpha + acc
    m_ref[...] = new_m

    u = trel_ref[...] - j * _BR
    p = p_ref[...]
    for q in range(_Q):
        p = jnp.where(u == 8 * q, x3[q], p)
    p_ref[...] = p

    @pl.when(j == _NBLK - 1)
    def _fin():
        m_row = jnp.max(m_ref[...], axis=0, keepdims=True)
        s_row = jnp.sum(s_ref[...] * jnp.exp(m_ref[...] - m_row),
                        axis=0, keepdims=True)
        picked = jnp.sum(p_ref[...], axis=0, keepdims=True)
        nll = m_row + jnp.log(s_row) - picked
        out_ref[...] = jnp.sum(nll, keepdims=True) / _ROWS


def _loss(x_t, trow):
    return pl.pallas_call(
        _loss_kernel,
        grid=(_NBLK,),
        in_specs=[
            pl.BlockSpec((_BR, _ROWS), lambda j: (j, 0)),
            pl.BlockSpec((1, _ROWS), lambda j: (0, 0)),
        ],
        out_specs=pl.BlockSpec((1, 1), lambda j: (0, 0)),
        out_shape=jax.ShapeDtypeStruct((1, 1), jnp.float32),
        scratch_shapes=[
            pltpu.VMEM((8, _ROWS), jnp.float32),
            pltpu.VMEM((8, _ROWS), jnp.float32),
            pltpu.VMEM((8, _ROWS), jnp.float32),
            pltpu.VMEM((8, _ROWS), jnp.int32),
        ],
    )(x_t, trow)


_NW = 32
_TPW = _ROWS // _NW


def _sc_weights_body(t2_hbm, out_hbm, t2_v, psum_v):
    cid = lax.axis_index("c")
    sid = lax.axis_index("s")
    wid = sid * 2 + cid
    pltpu.sync_copy(t2_hbm, t2_v)
    base = wid * _TPW
    tg1 = t2_v[pl.ds(base, 16)]
    tg2 = t2_v[pl.ds(base + 16, 16)]

    def body(off, cs):
        c1, c2 = cs
        ts = t2_v[pl.ds(off, 16)]
        one = jnp.int32(1)
        zero = jnp.int32(0)
        c1 = c1 + jnp.where(tg1 == ts, one, zero)
        c2 = c2 + jnp.where(tg2 == ts, one, zero)
        return (c1, c2)

    z = jnp.zeros((16,), jnp.int32)
    c1, c2 = lax.fori_loop(0, _ROWS, body, (z, z))

    def wsum(c):
        cf = c.astype(jnp.float32)
        return (1.0 - jnp.exp(cf * (_LN_BETA / _ROWS))) / cf

    psum_v[...] = wsum(c1) + wsum(c2)
    pltpu.sync_copy(psum_v, out_hbm.at[wid])


@functools.lru_cache(maxsize=None)
def _make_sc_weights():
    return pl.kernel(
        _sc_weights_body,
        mesh=plsc.VectorSubcoreMesh(core_axis_name="c", subcore_axis_name="s"),
        out_type=jax.ShapeDtypeStruct((_NW, 16), jnp.float32),
        scratch_types=[
            pltpu.VMEM((2 * _ROWS,), jnp.int32),
            pltpu.VMEM((16,), jnp.float32),
        ],
    )


def _sc_weights(t32):
    return _make_sc_weights()(jnp.concatenate([t32, t32]))


def kernel(inputs, targets):
    t32 = targets.astype(jnp.int32)
    loss = _loss(inputs.T, t32.reshape(1, _ROWS))[0, 0]
    wparts = _sc_weights(t32)
    return loss * (jnp.sum(wparts) / _NUM_CLASSES)


# speedup vs baseline: 1.0433x; 1.0130x over previous
import functools
import math

import jax
import jax.numpy as jnp
from jax import lax
from jax.experimental import pallas as pl
from jax.experimental.pallas import tpu as pltpu
from jax.experimental.pallas import tpu_sc as plsc

_NUM_CLASSES = 100000
_BETA = 0.8
_ROWS = 1024
_BR = 4000
_Q = _BR // 8
_NBLK = _NUM_CLASSES // _BR
_LN_BETA = math.log(_BETA)
_LOG2E = math.log2(math.e)
_NEG_INF = float("-inf")


def _loss_kernel(x_ref, trow_ref, out_ref, m_ref, s_ref, p_ref, trel_ref):
    j = pl.program_id(0)

    @pl.when(j == 0)
    def _init():
        m_ref[...] = jnp.full((8, _ROWS), _NEG_INF, jnp.float32)
        s_ref[...] = jnp.zeros((8, _ROWS), jnp.float32)
        p_ref[...] = jnp.zeros((8, _ROWS), jnp.float32)
        sub = jax.lax.broadcasted_iota(jnp.int32, (8, _ROWS), 0)
        trel_ref[...] = trow_ref[...] - sub

    x3 = x_ref[...].reshape(_Q, 8, _ROWS)
    tm = jnp.max(x3, axis=0)
    new_m = jnp.maximum(m_ref[...], tm)
    alpha = jnp.exp(m_ref[...] - new_m)
    nm2 = new_m * (-_LOG2E)
    acc = jnp.sum(jnp.exp2(x3 * _LOG2E + nm2[None]), axis=0)
    s_ref[...] = s_ref[...] * alpha + acc
    m_ref[...] = new_m

    u = trel_ref[...] - j * _BR
    p = p_ref[...]
    for q in range(_Q):
        p = jnp.where(u == 8 * q, x3[q], p)
    p_ref[...] = p

    @pl.when(j == _NBLK - 1)
    def _fin():
        m_row = jnp.max(m_ref[...], axis=0, keepdims=True)
        s_row = jnp.sum(s_ref[...] * jnp.exp(m_ref[...] - m_row),
                        axis=0, keepdims=True)
        picked = jnp.sum(p_ref[...], axis=0, keepdims=True)
        nll = m_row + jnp.log(s_row) - picked
        out_ref[...] = jnp.sum(nll, keepdims=True) / _ROWS


def _loss(x_t, trow):
    return pl.pallas_call(
        _loss_kernel,
        grid=(_NBLK,),
        in_specs=[
            pl.BlockSpec((_BR, _ROWS), lambda j: (j, 0)),
            pl.BlockSpec((1, _ROWS), lambda j: (0, 0)),
        ],
        out_specs=pl.BlockSpec((1, 1), lambda j: (0, 0)),
        out_shape=jax.ShapeDtypeStruct((1, 1), jnp.float32),
        scratch_shapes=[
            pltpu.VMEM((8, _ROWS), jnp.float32),
            pltpu.VMEM((8, _ROWS), jnp.float32),
            pltpu.VMEM((8, _ROWS), jnp.float32),
            pltpu.VMEM((8, _ROWS), jnp.int32),
        ],
    )(x_t, trow)


_NW = 32
_TPW = _ROWS // _NW


def _sc_weights_body(t_hbm, out_hbm, t2_v, psum_v):
    cid = lax.axis_index("c")
    sid = lax.axis_index("s")
    wid = sid * 2 + cid
    pltpu.sync_copy(t_hbm, t2_v.at[pl.ds(0, _ROWS)])
    pltpu.sync_copy(t_hbm, t2_v.at[pl.ds(_ROWS, _ROWS)])
    base = wid * _TPW
    tg1 = t2_v[pl.ds(base, 16)]
    tg2 = t2_v[pl.ds(base + 16, 16)]

    def body(off, cs):
        c1, c2 = cs
        ts = t2_v[pl.ds(off, 16)]
        one = jnp.int32(1)
        zero = jnp.int32(0)
        c1 = c1 + jnp.where(tg1 == ts, one, zero)
        c2 = c2 + jnp.where(tg2 == ts, one, zero)
        return (c1, c2)

    z = jnp.zeros((16,), jnp.int32)
    c1, c2 = lax.fori_loop(0, _ROWS, body, (z, z))

    def wsum(c):
        cf = c.astype(jnp.float32)
        return (1.0 - jnp.exp(cf * (_LN_BETA / _ROWS))) / cf

    psum_v[...] = wsum(c1) + wsum(c2)
    pltpu.sync_copy(psum_v, out_hbm.at[wid])


@functools.lru_cache(maxsize=None)
def _make_sc_weights():
    return pl.kernel(
        _sc_weights_body,
        mesh=plsc.VectorSubcoreMesh(core_axis_name="c", subcore_axis_name="s"),
        out_type=jax.ShapeDtypeStruct((_NW, 16), jnp.float32),
        scratch_types=[
            pltpu.VMEM((2 * _ROWS,), jnp.int32),
            pltpu.VMEM((16,), jnp.float32),
        ],
    )


def _sc_weights(t32):
    return _make_sc_weights()(t32)


def kernel(inputs, targets):
    t32 = targets.astype(jnp.int32)
    loss = _loss(inputs.T, t32.reshape(1, _ROWS))[0, 0]
    wparts = _sc_weights(t32)
    return loss * (jnp.sum(wparts) / _NUM_CLASSES)
